# baseline (speedup 1.0000x reference)
import jax
import jax.numpy as jnp
from jax import lax
from jax.experimental import pallas as pl
from jax.experimental.pallas import tpu as pltpu

NBUF = 4
CB = 16


def _body(x_hbm, p_vmem, o_hbm, bufs, obufs, in_sems, out_sems):
    nchunk = x_hbm.shape[0] // CB
    pos = p_vmem[...]

    def in_copy(chunk, slot):
        return pltpu.make_async_copy(
            x_hbm.at[pl.ds(chunk * CB, CB)], bufs.at[slot], in_sems.at[slot])

    def out_copy(chunk, slot):
        return pltpu.make_async_copy(
            obufs.at[slot], o_hbm.at[pl.ds(chunk * CB, CB)], out_sems.at[slot])

    for s in range(NBUF):
        in_copy(s, s).start()

    def step(c, _):
        slot = lax.rem(c, NBUF)
        in_copy(c, slot).wait()

        @pl.when(c >= NBUF)
        def _wait_prev_out():
            out_copy(c - NBUF, slot).wait()

        obufs[slot] = bufs[slot] + pos
        out_copy(c, slot).start()

        @pl.when(c + NBUF < nchunk)
        def _start_next_in():
            in_copy(c + NBUF, slot).start()

        return _

    lax.fori_loop(0, nchunk, step, None)
    for s in range(NBUF):
        c = nchunk - NBUF + s
        out_copy(c, c % NBUF).wait()


def kernel(x, pos_table):
    B, M, D = x.shape
    xt = jnp.transpose(x, (0, 2, 1))
    pt = jnp.transpose(pos_table, (1, 0))
    out_t = pl.pallas_call(
        _body,
        in_specs=[
            pl.BlockSpec(memory_space=pltpu.MemorySpace.HBM),
            pl.BlockSpec(memory_space=pltpu.MemorySpace.VMEM),
        ],
        out_specs=pl.BlockSpec(memory_space=pltpu.MemorySpace.HBM),
        out_shape=jax.ShapeDtypeStruct((B, D, M), x.dtype),
        scratch_shapes=[
            pltpu.VMEM((NBUF, CB, D, M), jnp.float32),
            pltpu.VMEM((NBUF, CB, D, M), jnp.float32),
            pltpu.SemaphoreType.DMA((NBUF,)),
            pltpu.SemaphoreType.DMA((NBUF,)),
        ],
    )(xt, pt)
    return jnp.transpose(out_t, (0, 2, 1))


# speedup vs baseline: 1.0003x; 1.0003x over previous
import jax
import jax.numpy as jnp
from jax.experimental import pallas as pl

CB = 32


def _add_body(x_ref, p_ref, o_ref):
    o_ref[...] = x_ref[...] + p_ref[...]


def kernel(x, pos_table):
    B, M, D = x.shape
    xt = jnp.transpose(x, (0, 2, 1))
    pt = jnp.transpose(pos_table, (1, 0))
    out_t = pl.pallas_call(
        _add_body,
        grid=(B // CB,),
        in_specs=[
            pl.BlockSpec((CB, D, M), lambda i: (i, 0, 0)),
            pl.BlockSpec((D, M), lambda i: (0, 0)),
        ],
        out_specs=pl.BlockSpec((CB, D, M), lambda i: (i, 0, 0)),
        out_shape=jax.ShapeDtypeStruct((B, D, M), x.dtype),
    )(xt, pt)
    return jnp.transpose(out_t, (0, 2, 1))
